# baseline (speedup 1.0000x reference)
import functools

import jax
import jax.numpy as jnp
from jax import lax
from jax.experimental import pallas as pl
from jax.experimental.pallas import tpu as pltpu
from jax.experimental.pallas import tpu_sc as plsc

_NC, _NS = 2, 16
_NW = _NC * _NS
_D = 64
_B = 4096 * 200
_BPW = _B // _NW
_C = 512
_G = 128
_NCHUNK = _BPW // _C


def _body(table_hbm, idx_hbm, out_hbm, idx_v, rows_v, sem):
    wid = lax.axis_index("s") * _NC + lax.axis_index("c")
    base = wid * _BPW

    def chunk(i, carry):
        off = base + i * _C
        pltpu.sync_copy(idx_hbm.at[pl.ds(off, _C)], idx_v)
        cps = []
        for j in range(_C // _G):
            cps.append(
                pltpu.async_copy(
                    table_hbm.at[idx_v.at[pl.ds(j * _G, _G)]],
                    rows_v.at[pl.ds(j * _G, _G)],
                    sem,
                )
            )
        for cp in cps:
            cp.wait()
        pltpu.sync_copy(rows_v, out_hbm.at[pl.ds(off, _C)])
        return carry

    lax.fori_loop(0, _NCHUNK, chunk, 0)


@jax.jit
def _gather(table, flat_idx):
    mesh = plsc.VectorSubcoreMesh(core_axis_name="c", subcore_axis_name="s")
    f = pl.kernel(
        _body,
        out_type=jax.ShapeDtypeStruct((_B, _D), jnp.float32),
        mesh=mesh,
        scratch_types=[
            pltpu.VMEM((_C,), jnp.int32),
            pltpu.VMEM((_C, _D), jnp.float32),
            pltpu.SemaphoreType.DMA,
        ],
        compiler_params=pltpu.CompilerParams(use_tc_tiling_on_sc=False),
    )
    return f(table, flat_idx)


def kernel(input, table):
    flat = input.reshape(-1)
    out = _gather(table, flat)
    return out.reshape(input.shape + (_D,))


# speedup vs baseline: 1.0423x; 1.0423x over previous
import functools

import jax
import jax.numpy as jnp
from jax import lax
from jax.experimental import pallas as pl
from jax.experimental.pallas import tpu as pltpu
from jax.experimental.pallas import tpu_sc as plsc

_NC, _NS = 2, 16
_NW = _NC * _NS
_D = 64
_B = 4096 * 200
_BPW = _B // _NW
_C = 640
_G = 128
_GPC = _C // _G
_NCHUNK = _BPW // _C
_PAIRS = _NCHUNK // 2


def _body(table_hbm, idx_hbm, out_hbm, idx_v, rows0, rows1,
          sem_g0, sem_g1, sem_o0, sem_o1):
    wid = lax.axis_index("s") * _NC + lax.axis_index("c")
    base = wid * _BPW

    pltpu.sync_copy(idx_hbm.at[pl.ds(base, _BPW)], idx_v)

    rows = (rows0, rows1)
    sem_g = (sem_g0, sem_g1)
    sem_o = (sem_o0, sem_o1)

    def fire_gathers(c, slot):
        for j in range(_GPC):
            pltpu.async_copy(
                table_hbm.at[idx_v.at[pl.ds(c * _C + j * _G, _G)]],
                rows[slot].at[pl.ds(j * _G, _G)],
                sem_g[slot],
            )

    def drain_gathers(slot):
        for j in range(_GPC):
            pltpu.make_async_copy(
                table_hbm.at[idx_v.at[pl.ds(j * _G, _G)]],
                rows[slot].at[pl.ds(j * _G, _G)],
                sem_g[slot],
            ).wait()

    def start_out(c, slot):
        pltpu.async_copy(
            rows[slot], out_hbm.at[pl.ds(base + c * _C, _C)], sem_o[slot]
        )

    def drain_out(slot):
        pltpu.make_async_copy(
            rows[slot], out_hbm.at[pl.ds(base, _C)], sem_o[slot]
        ).wait()

    def pair(t, carry):
        a = 2 * t
        b = a + 1

        @pl.when(t > 0)
        def _():
            drain_out(0)

        fire_gathers(a, 0)

        @pl.when(t > 0)
        def _():
            drain_out(1)

        drain_gathers(0)
        start_out(a, 0)
        fire_gathers(b, 1)
        drain_gathers(1)
        start_out(b, 1)
        return carry

    lax.fori_loop(0, _PAIRS, pair, 0)
    drain_out(0)
    drain_out(1)


@jax.jit
def _gather(table, flat_idx):
    mesh = plsc.VectorSubcoreMesh(core_axis_name="c", subcore_axis_name="s")
    f = pl.kernel(
        _body,
        out_type=jax.ShapeDtypeStruct((_B, _D), jnp.float32),
        mesh=mesh,
        scratch_types=[
            pltpu.VMEM((_BPW,), jnp.int32),
            pltpu.VMEM((_C, _D), jnp.float32),
            pltpu.VMEM((_C, _D), jnp.float32),
            pltpu.SemaphoreType.DMA,
            pltpu.SemaphoreType.DMA,
            pltpu.SemaphoreType.DMA,
            pltpu.SemaphoreType.DMA,
        ],
        compiler_params=pltpu.CompilerParams(use_tc_tiling_on_sc=False),
    )
    return f(table, flat_idx)


def kernel(input, table):
    flat = input.reshape(-1)
    out = _gather(table, flat)
    return out.reshape(input.shape + (_D,))
